# speedup vs baseline: 2.1214x; 2.1214x over previous
import functools

import jax
import jax.numpy as jnp
from jax import lax
from jax.experimental import pallas as pl
from jax.experimental.pallas import tpu as pltpu
from jax.experimental.pallas import tpu_sc as plsc

SEQ_LEN = 200
NUM_HID = 128
BATCH = 4096
TOTAL = BATCH * SEQ_LEN
NUM_WORKERS = 32
PER_WORKER = TOTAL // NUM_WORKERS
CHUNK = SEQ_LEN
N_CHUNKS = PER_WORKER // CHUNK
LANES = 16
SLICES = NUM_HID // LANES


def _pos_encode(seq_len, dim):
    range_even = jnp.arange(dim, dtype=jnp.float32)
    range_even = (range_even / 2.0).astype(jnp.int32).astype(jnp.float32)
    power = range_even / float(dim)
    denom = jnp.power(10000.0, power).reshape(1, dim)
    pos = jnp.arange(seq_len, dtype=jnp.float32).reshape(seq_len, 1)
    arg = pos / denom
    cos_mask = (jnp.arange(seq_len) % 2).astype(bool).reshape(seq_len, 1)
    sin = jnp.where(jnp.logical_not(cos_mask), jnp.sin(arg), 0.0)
    cos = jnp.where(cos_mask, jnp.cos(arg), 0.0)
    return sin + cos


@functools.partial(
    pl.kernel,
    out_type=jax.ShapeDtypeStruct((TOTAL, NUM_HID), jnp.float32),
    mesh=plsc.VectorSubcoreMesh(core_axis_name="c", subcore_axis_name="s"),
    scratch_types=[
        pltpu.VMEM((CHUNK,), jnp.int32),
        pltpu.VMEM((CHUNK, NUM_HID), jnp.float32),
        pltpu.VMEM((SEQ_LEN, NUM_HID), jnp.float32),
        pltpu.SemaphoreType.DMA,
    ],
)
def _embed_sc(idx_hbm, table_hbm, pe_hbm, out_hbm, idx_v, rows_v, pe_v, sem):
    wid = lax.axis_index("s") * 2 + lax.axis_index("c")
    base = wid * PER_WORKER
    pltpu.sync_copy(pe_hbm, pe_v)

    def chunk_body(c, carry):
        off = base + c * CHUNK
        pltpu.sync_copy(idx_hbm.at[pl.ds(off, CHUNK)], idx_v)
        pltpu.async_copy(table_hbm.at[idx_v], rows_v, sem).wait()

        def row_body(r, rcarry):
            for k in range(SLICES):
                sl = pl.ds(k * LANES, LANES)
                rows_v[r, sl] = rows_v[r, sl] + pe_v[r, sl]
            return rcarry

        lax.fori_loop(0, CHUNK, row_body, 0, unroll=2)
        pltpu.sync_copy(rows_v, out_hbm.at[pl.ds(off, CHUNK)])
        return carry

    lax.fori_loop(0, N_CHUNKS, chunk_body, 0)


def kernel(input, table):
    idx = input.reshape(TOTAL).astype(jnp.int32)
    pe = _pos_encode(SEQ_LEN, NUM_HID)
    out = _embed_sc(idx, table, pe)
    return out.reshape(BATCH, SEQ_LEN, NUM_HID)
